# baseline (speedup 1.0000x reference)
import functools

import jax
import jax.numpy as jnp
from jax import lax
from jax.experimental import pallas as pl
from jax.experimental.pallas import tpu as pltpu
from jax.experimental.pallas import tpu_sc as plsc

_EPS = 1e-06
_CH = 2048
_BIG = 1e9
_NW = 32


def _tc_body(gtb_ref, glab_ref, anch_ref, gidx_ref, posf_ref, gtab_ref,
             *, n_chunks):
    gtb = gtb_ref[0]
    gx1 = gtb[:, 0:1]
    gy1 = gtb[:, 1:2]
    gx2 = gtb[:, 2:3]
    gy2 = gtb[:, 3:4]
    area_g = jnp.clip(gx2 - gx1, 0.0) * jnp.clip(gy2 - gy1, 0.0)
    glab = glab_ref[0]
    n_iota = lax.broadcasted_iota(jnp.int32, (64, 1), 0).astype(jnp.float32)

    gtab_ref[0] = jnp.concatenate(
        [gx1, gy1, jnp.log(gx2 + _EPS), jnp.log(gy2 + _EPS),
         glab, glab, glab, glab], axis=1)

    def iou_chunk(c):
        sl = pl.ds(c * _CH, _CH)
        acx = anch_ref[0:1, sl]
        acy = anch_ref[1:2, sl]
        aw = anch_ref[2:3, sl]
        ah = anch_ref[3:4, sl]
        ax1 = acx - aw * 0.5
        ay1 = acy - ah * 0.5
        ax2 = acx + aw * 0.5
        ay2 = acy + ah * 0.5
        ltx = jnp.maximum(ax1, gx1)
        lty = jnp.maximum(ay1, gy1)
        rbx = jnp.minimum(ax2, gx2)
        rby = jnp.minimum(ay2, gy2)
        w = jnp.clip(rbx - ltx, 0.0)
        h = jnp.clip(rby - lty, 0.0)
        inter = w * h
        area_a = jnp.clip(ax2 - ax1, 0.0) * jnp.clip(ay2 - ay1, 0.0)
        union = area_a + area_g - inter
        return inter / union

    run_max = jnp.full((64, 1), -1.0, jnp.float32)
    run_arg = jnp.zeros((64, 1), jnp.float32)
    rows = []
    for c in range(n_chunks):
        iou = iou_chunk(c)
        row_max = jnp.max(iou, axis=0, keepdims=True)
        row_arg = jnp.min(jnp.where(iou == row_max, n_iota, _BIG),
                          axis=0, keepdims=True)
        rows.append((row_max, row_arg))
        a_iota = (lax.broadcasted_iota(jnp.int32, (1, _CH), 1).astype(jnp.float32)
                  + float(c * _CH))
        col_max = jnp.max(iou, axis=1, keepdims=True)
        col_arg = jnp.min(jnp.where(iou == col_max, a_iota, _BIG),
                          axis=1, keepdims=True)
        upd = col_max > run_max
        run_max = jnp.where(upd, col_max, run_max)
        run_arg = jnp.where(upd, col_arg, run_arg)
    best_anchor = run_arg

    b64 = (pl.program_id(0) * 64).astype(jnp.float32)
    for c in range(n_chunks):
        a_iota = (lax.broadcasted_iota(jnp.int32, (1, _CH), 1).astype(jnp.float32)
                  + float(c * _CH))
        eq = best_anchor == a_iota
        n_sel = jnp.max(jnp.where(eq, n_iota, -1.0), axis=0, keepdims=True)
        ovr = n_sel >= 0.0
        biou, bidx = rows[c]
        fidx = jnp.where(ovr, n_sel, bidx)
        fiou = jnp.where(ovr, 2.0, biou)
        pos = fiou > 0.5
        gidx_ref[0, c, :] = (fidx + b64
                             + jnp.where(pos, 512.0, 0.0)).astype(jnp.int32)[0]
        posf_ref[0, c, :] = pos.astype(jnp.float32)[0]


def _sc_encode_body(gidx_h, table_h, aint_h, enc_h, lab_h,
                    table_v, gidx_v, aint_v, rows_v, lab_v, *, per_w, a_pad):
    ncores = plsc.get_sparse_core_info().num_cores
    wid = lax.axis_index("s") * ncores + lax.axis_index("c")
    base = wid * per_w
    la = (wid % (a_pad // per_w)) * per_w

    pltpu.sync_copy(table_h, table_v)
    pltpu.sync_copy(gidx_h.at[pl.ds(base, per_w)], gidx_v)
    pltpu.sync_copy(aint_h.at[pl.ds(la * 8, per_w * 8)], aint_v)

    iota16 = lax.broadcasted_iota(jnp.int32, (16,), 0)

    def body(i, carry):
        sl = pl.ds(i * 16, 16)
        v = gidx_v[sl]
        posf = (v >> 9).astype(jnp.float32)
        idx8 = (v & 511) * 8
        m0 = plsc.load_gather(table_v, [idx8])
        m1 = plsc.load_gather(table_v, [idx8 + 1])
        lg2 = plsc.load_gather(table_v, [idx8 + 2])
        lg3 = plsc.load_gather(table_v, [idx8 + 3])
        mlab = plsc.load_gather(table_v, [idx8 + 4])
        aidx8 = (iota16 + i * 16) * 8
        acx = plsc.load_gather(aint_v, [aidx8])
        acy = plsc.load_gather(aint_v, [aidx8 + 1])
        aw = plsc.load_gather(aint_v, [aidx8 + 2])
        ah = plsc.load_gather(aint_v, [aidx8 + 3])
        law = plsc.load_gather(aint_v, [aidx8 + 4])
        lah = plsc.load_gather(aint_v, [aidx8 + 5])
        e0 = (m0 - acx) / aw
        e1 = (m1 - acy) / ah
        e2 = lg2 - law
        e3 = lg3 - lah
        lab = (posf * mlab).astype(jnp.int32)
        ridx4 = (iota16 + i * 16) * 4
        plsc.store_scatter(rows_v, [ridx4], e0)
        plsc.store_scatter(rows_v, [ridx4 + 1], e1)
        plsc.store_scatter(rows_v, [ridx4 + 2], e2)
        plsc.store_scatter(rows_v, [ridx4 + 3], e3)
        lab_v[sl] = lab
        return carry

    lax.fori_loop(0, per_w // 16, body, 0)

    pltpu.sync_copy(rows_v, enc_h.at[pl.ds(base * 4, per_w * 4)])
    pltpu.sync_copy(lab_v, lab_h.at[pl.ds(base, per_w)])


def kernel(gt_boxes, gt_labels, mask, anchors):
    del mask
    B, N, _ = gt_boxes.shape
    A = anchors.shape[0]
    n_chunks = -(-A // _CH)
    A_pad = n_chunks * _CH
    pad_row = jnp.array([-10.0, -10.0, 1.0, 1.0], jnp.float32)
    anchors_p = jnp.concatenate(
        [anchors, jnp.broadcast_to(pad_row, (A_pad - A, 4))], axis=0)
    anchors_t = anchors_p.T
    glab = gt_labels.astype(jnp.float32)[..., None]
    aw = anchors_p[:, 2]
    ah = anchors_p[:, 3]
    zero = jnp.zeros_like(aw)
    ainter = jnp.stack(
        [anchors_p[:, 0], anchors_p[:, 1], aw, ah,
         jnp.log(aw + _EPS), jnp.log(ah + _EPS), zero, zero],
        axis=1).reshape(A_pad * 8)

    tc_body = functools.partial(_tc_body, n_chunks=n_chunks)
    gidx, posf, gtab = pl.pallas_call(
        tc_body,
        grid=(B,),
        in_specs=[
            pl.BlockSpec((1, N, 4), lambda b: (b, 0, 0)),
            pl.BlockSpec((1, N, 1), lambda b: (b, 0, 0)),
            pl.BlockSpec((4, A_pad), lambda b: (0, 0)),
        ],
        out_specs=[
            pl.BlockSpec((1, n_chunks, _CH), lambda b: (b, 0, 0)),
            pl.BlockSpec((1, n_chunks, _CH), lambda b: (b, 0, 0)),
            pl.BlockSpec((1, N, 8), lambda b: (b, 0, 0)),
        ],
        out_shape=[
            jax.ShapeDtypeStruct((B, n_chunks, _CH), jnp.int32),
            jax.ShapeDtypeStruct((B, n_chunks, _CH), jnp.float32),
            jax.ShapeDtypeStruct((B, N, 8), jnp.float32),
        ],
    )(gt_boxes, glab, anchors_t)

    total = B * A_pad
    per_w = total // _NW
    sc_body = functools.partial(_sc_encode_body, per_w=per_w, a_pad=A_pad)
    sc_encode = pl.kernel(
        sc_body,
        out_type=[
            jax.ShapeDtypeStruct((total * 4,), jnp.float32),
            jax.ShapeDtypeStruct((total,), jnp.int32),
        ],
        mesh=plsc.VectorSubcoreMesh(core_axis_name="c", subcore_axis_name="s"),
        compiler_params=pltpu.CompilerParams(needs_layout_passes=False),
        scratch_types=[
            pltpu.VMEM((B * N * 8,), jnp.float32),
            pltpu.VMEM((per_w,), jnp.int32),
            pltpu.VMEM((per_w * 8,), jnp.float32),
            pltpu.VMEM((per_w * 4,), jnp.float32),
            pltpu.VMEM((per_w,), jnp.int32),
        ],
    )
    enc_f, lab_f = sc_encode(
        gidx.reshape(total), gtab.reshape(B * N * 8), ainter)

    encoded = enc_f.reshape(B, A_pad, 4)[:, :A, :]
    encoded_labels = lab_f.reshape(B, A_pad)[:, :A]
    pos_mask = posf.reshape(B, A_pad)[:, :A] > 0.5
    return (encoded, encoded_labels, pos_mask)


# speedup vs baseline: 3.8293x; 3.8293x over previous
import jax
import jax.numpy as jnp
from jax import lax
from jax.experimental import pallas as pl
from jax.experimental.pallas import tpu as pltpu

_EPS = 1e-06
_BACKGROUND = 0.0
_CH = 2048
_BIG = 1e9


def _body(gtb_ref, glab_ref, anch_ref, enc_ref, lab_ref, pos_ref,
          *, n_chunks):
    gtb = gtb_ref[0]
    gx1 = gtb[:, 0:1]
    gy1 = gtb[:, 1:2]
    gx2 = gtb[:, 2:3]
    gy2 = gtb[:, 3:4]
    area_g = jnp.clip(gx2 - gx1, 0.0) * jnp.clip(gy2 - gy1, 0.0)
    glab = glab_ref[0]
    n_iota = lax.broadcasted_iota(jnp.int32, (64, 1), 0).astype(jnp.float32)

    def anchor_chunk(c):
        sl = pl.ds(c * _CH, _CH)
        acx = anch_ref[0:1, sl]
        acy = anch_ref[1:2, sl]
        aw = anch_ref[2:3, sl]
        ah = anch_ref[3:4, sl]
        return acx, acy, aw, ah

    def iou_chunk(c):
        acx, acy, aw, ah = anchor_chunk(c)
        ax1 = acx - aw * 0.5
        ay1 = acy - ah * 0.5
        ax2 = acx + aw * 0.5
        ay2 = acy + ah * 0.5
        ltx = jnp.maximum(ax1, gx1)
        lty = jnp.maximum(ay1, gy1)
        rbx = jnp.minimum(ax2, gx2)
        rby = jnp.minimum(ay2, gy2)
        w = jnp.clip(rbx - ltx, 0.0)
        h = jnp.clip(rby - lty, 0.0)
        inter = w * h
        area_a = jnp.clip(ax2 - ax1, 0.0) * jnp.clip(ay2 - ay1, 0.0)
        union = area_a + area_g - inter
        return inter / union

    run_max = jnp.full((64, 1), -1.0, jnp.float32)
    run_arg = jnp.zeros((64, 1), jnp.float32)
    rows = []
    for c in range(n_chunks):
        iou = iou_chunk(c)
        row_max = jnp.max(iou, axis=0, keepdims=True)
        row_arg = jnp.min(jnp.where(iou == row_max, n_iota, _BIG),
                          axis=0, keepdims=True)
        rows.append((row_max, row_arg))
        a_iota = (lax.broadcasted_iota(jnp.int32, (1, _CH), 1).astype(jnp.float32)
                  + float(c * _CH))
        col_max = jnp.max(iou, axis=1, keepdims=True)
        col_arg = jnp.min(jnp.where(iou == col_max, a_iota, _BIG),
                          axis=1, keepdims=True)
        upd = col_max > run_max
        run_max = jnp.where(upd, col_max, run_max)
        run_arg = jnp.where(upd, col_arg, run_arg)
    best_anchor = run_arg

    for c in range(n_chunks):
        acx, acy, aw, ah = anchor_chunk(c)
        a_iota = (lax.broadcasted_iota(jnp.int32, (1, _CH), 1).astype(jnp.float32)
                  + float(c * _CH))
        eq = best_anchor == a_iota
        n_sel = jnp.max(jnp.where(eq, n_iota, -1.0), axis=0, keepdims=True)
        ovr = n_sel >= 0.0
        biou, bidx = rows[c]
        fidx = jnp.where(ovr, n_sel, bidx)
        fiou = jnp.where(ovr, 2.0, biou)
        pos = fiou > 0.5
        oh = (n_iota == fidx).astype(jnp.float32)
        gmat = jnp.concatenate(
            [gx1, gy1, gx2, gy2, glab, glab, glab, glab], axis=1)
        m = lax.dot_general(gmat, oh, (((0,), (0,)), ((), ())),
                            preferred_element_type=jnp.float32,
                            precision=lax.Precision.HIGHEST)
        m0 = m[0:1]
        m1 = m[1:2]
        m2 = m[2:3]
        m3 = m[3:4]
        mlab = m[4:5]
        e0 = (m0 - acx) / aw
        e1 = (m1 - acy) / ah
        e2 = jnp.log((m2 + _EPS) / (aw + _EPS))
        e3 = jnp.log((m3 + _EPS) / (ah + _EPS))
        enc_ref[0, 0, c, :] = e0[0]
        enc_ref[0, 1, c, :] = e1[0]
        enc_ref[0, 2, c, :] = e2[0]
        enc_ref[0, 3, c, :] = e3[0]
        lab_ref[0, c, :] = jnp.where(pos, mlab, _BACKGROUND)[0]
        pos_ref[0, c, :] = pos.astype(jnp.float32)[0]


def kernel(gt_boxes, gt_labels, mask, anchors):
    del mask
    B, N, _ = gt_boxes.shape
    A = anchors.shape[0]
    n_chunks = -(-A // _CH)
    A_pad = n_chunks * _CH
    pad_row = jnp.array([-10.0, -10.0, 1.0, 1.0], jnp.float32)
    anchors_p = jnp.concatenate(
        [anchors, jnp.broadcast_to(pad_row, (A_pad - A, 4))], axis=0)
    anchors_t = anchors_p.T
    glab = gt_labels.astype(jnp.float32)[..., None]

    import functools
    body = functools.partial(_body, n_chunks=n_chunks)
    enc, lab, pos = pl.pallas_call(
        body,
        grid=(B,),
        in_specs=[
            pl.BlockSpec((1, N, 4), lambda b: (b, 0, 0)),
            pl.BlockSpec((1, N, 1), lambda b: (b, 0, 0)),
            pl.BlockSpec((4, A_pad), lambda b: (0, 0)),
        ],
        out_specs=[
            pl.BlockSpec((1, 4, n_chunks, _CH), lambda b: (b, 0, 0, 0)),
            pl.BlockSpec((1, n_chunks, _CH), lambda b: (b, 0, 0)),
            pl.BlockSpec((1, n_chunks, _CH), lambda b: (b, 0, 0)),
        ],
        out_shape=[
            jax.ShapeDtypeStruct((B, 4, n_chunks, _CH), jnp.float32),
            jax.ShapeDtypeStruct((B, n_chunks, _CH), jnp.float32),
            jax.ShapeDtypeStruct((B, n_chunks, _CH), jnp.float32),
        ],
    )(gt_boxes, glab, anchors_t)

    encoded = enc.reshape(B, 4, A_pad)[:, :, :A].transpose(0, 2, 1)
    encoded_labels = lab.reshape(B, A_pad)[:, :A].astype(jnp.int32)
    pos_mask = pos.reshape(B, A_pad)[:, :A] > 0.5
    return (encoded, encoded_labels, pos_mask)


# speedup vs baseline: 3.9579x; 1.0336x over previous
import jax
import jax.numpy as jnp
from jax import lax
from jax.experimental import pallas as pl
from jax.experimental.pallas import tpu as pltpu

_EPS = 1e-06
_BACKGROUND = 0.0
_CH = 1024
_BIG = 1e9


def _body(gtb_ref, glab_ref, anch_ref, enc_ref, lab_ref, pos_ref,
          *, n_chunks):
    gtb = gtb_ref[0]
    gx1 = gtb[:, 0:1]
    gy1 = gtb[:, 1:2]
    gx2 = gtb[:, 2:3]
    gy2 = gtb[:, 3:4]
    area_g = jnp.clip(gx2 - gx1, 0.0) * jnp.clip(gy2 - gy1, 0.0)
    glab = glab_ref[0]
    n_iota = lax.broadcasted_iota(jnp.int32, (64, 1), 0).astype(jnp.float32)

    def anchor_chunk(c):
        sl = pl.ds(c * _CH, _CH)
        acx = anch_ref[0:1, sl]
        acy = anch_ref[1:2, sl]
        aw = anch_ref[2:3, sl]
        ah = anch_ref[3:4, sl]
        return acx, acy, aw, ah

    def iou_chunk(c):
        acx, acy, aw, ah = anchor_chunk(c)
        ax1 = acx - aw * 0.5
        ay1 = acy - ah * 0.5
        ax2 = acx + aw * 0.5
        ay2 = acy + ah * 0.5
        ltx = jnp.maximum(ax1, gx1)
        lty = jnp.maximum(ay1, gy1)
        rbx = jnp.minimum(ax2, gx2)
        rby = jnp.minimum(ay2, gy2)
        w = jnp.clip(rbx - ltx, 0.0)
        h = jnp.clip(rby - lty, 0.0)
        inter = w * h
        area_a = jnp.clip(ax2 - ax1, 0.0) * jnp.clip(ay2 - ay1, 0.0)
        union = area_a + area_g - inter
        return inter / union

    run_max = jnp.full((64, 1), -1.0, jnp.float32)
    run_arg = jnp.zeros((64, 1), jnp.float32)
    rows = []
    for c in range(n_chunks):
        iou = iou_chunk(c)
        row_max = jnp.max(iou, axis=0, keepdims=True)
        row_arg = jnp.min(jnp.where(iou == row_max, n_iota, _BIG),
                          axis=0, keepdims=True)
        rows.append((row_max, row_arg))
        a_iota = (lax.broadcasted_iota(jnp.int32, (1, _CH), 1).astype(jnp.float32)
                  + float(c * _CH))
        col_max = jnp.max(iou, axis=1, keepdims=True)
        col_arg = jnp.min(jnp.where(iou == col_max, a_iota, _BIG),
                          axis=1, keepdims=True)
        upd = col_max > run_max
        run_max = jnp.where(upd, col_max, run_max)
        run_arg = jnp.where(upd, col_arg, run_arg)
    best_anchor = run_arg

    for c in range(n_chunks):
        acx, acy, aw, ah = anchor_chunk(c)
        a_iota = (lax.broadcasted_iota(jnp.int32, (1, _CH), 1).astype(jnp.float32)
                  + float(c * _CH))
        eq = best_anchor == a_iota
        n_sel = jnp.max(jnp.where(eq, n_iota, -1.0), axis=0, keepdims=True)
        ovr = n_sel >= 0.0
        biou, bidx = rows[c]
        fidx = jnp.where(ovr, n_sel, bidx)
        fiou = jnp.where(ovr, 2.0, biou)
        pos = fiou > 0.5
        oh = (n_iota == fidx).astype(jnp.float32)
        gmat = jnp.concatenate(
            [gx1, gy1, gx2, gy2, glab, glab, glab, glab], axis=1)
        m = lax.dot_general(gmat, oh, (((0,), (0,)), ((), ())),
                            preferred_element_type=jnp.float32,
                            precision=lax.Precision.HIGHEST)
        m0 = m[0:1]
        m1 = m[1:2]
        m2 = m[2:3]
        m3 = m[3:4]
        mlab = m[4:5]
        e0 = (m0 - acx) / aw
        e1 = (m1 - acy) / ah
        e2 = jnp.log((m2 + _EPS) / (aw + _EPS))
        e3 = jnp.log((m3 + _EPS) / (ah + _EPS))
        enc_ref[0, 0, c, :] = e0[0]
        enc_ref[0, 1, c, :] = e1[0]
        enc_ref[0, 2, c, :] = e2[0]
        enc_ref[0, 3, c, :] = e3[0]
        lab_ref[0, c, :] = jnp.where(pos, mlab, _BACKGROUND)[0]
        pos_ref[0, c, :] = pos.astype(jnp.float32)[0]


def kernel(gt_boxes, gt_labels, mask, anchors):
    del mask
    B, N, _ = gt_boxes.shape
    A = anchors.shape[0]
    n_chunks = -(-A // _CH)
    A_pad = n_chunks * _CH
    pad_row = jnp.array([-10.0, -10.0, 1.0, 1.0], jnp.float32)
    anchors_p = jnp.concatenate(
        [anchors, jnp.broadcast_to(pad_row, (A_pad - A, 4))], axis=0)
    anchors_t = anchors_p.T
    glab = gt_labels.astype(jnp.float32)[..., None]

    import functools
    body = functools.partial(_body, n_chunks=n_chunks)
    enc, lab, pos = pl.pallas_call(
        body,
        grid=(B,),
        in_specs=[
            pl.BlockSpec((1, N, 4), lambda b: (b, 0, 0)),
            pl.BlockSpec((1, N, 1), lambda b: (b, 0, 0)),
            pl.BlockSpec((4, A_pad), lambda b: (0, 0)),
        ],
        out_specs=[
            pl.BlockSpec((1, 4, n_chunks, _CH), lambda b: (b, 0, 0, 0)),
            pl.BlockSpec((1, n_chunks, _CH), lambda b: (b, 0, 0)),
            pl.BlockSpec((1, n_chunks, _CH), lambda b: (b, 0, 0)),
        ],
        out_shape=[
            jax.ShapeDtypeStruct((B, 4, n_chunks, _CH), jnp.float32),
            jax.ShapeDtypeStruct((B, n_chunks, _CH), jnp.float32),
            jax.ShapeDtypeStruct((B, n_chunks, _CH), jnp.float32),
        ],
    )(gt_boxes, glab, anchors_t)

    encoded = enc.reshape(B, 4, A_pad)[:, :, :A].transpose(0, 2, 1)
    encoded_labels = lab.reshape(B, A_pad)[:, :A].astype(jnp.int32)
    pos_mask = pos.reshape(B, A_pad)[:, :A] > 0.5
    return (encoded, encoded_labels, pos_mask)
